# Initial kernel scaffold; baseline (speedup 1.0000x reference)
import functools

import jax
import jax.numpy as jnp
from jax.experimental import pallas as pl
from jax.experimental.pallas import tpu as pltpu
from jax.experimental.pallas import tpu_sc as plsc

E = 16
H = 1024
I = 1024
M = 256


def _route(topk_indices, topk_weights, tiles):
    T_, K_ = topk_indices.shape
    N = T_ * K_
    pad_n = tiles * M
    e_flat = topk_indices.reshape(N).astype(jnp.int32)
    order = jnp.argsort(e_flat, stable=True).astype(jnp.int32)
    sorted_e = e_flat[order]
    g = jnp.zeros((E,), jnp.int32).at[e_flat].add(1)
    seg_start = (jnp.cumsum(g) - g).astype(jnp.int32)
    padded = ((g + (M - 1)) // M) * M
    ends = jnp.cumsum(padded).astype(jnp.int32)
    base = ends - padded
    s = jnp.arange(N, dtype=jnp.int32)
    rank = s - seg_start[sorted_e]
    dest = base[sorted_e] + rank
    tok = order // K_
    token_for_slot = jnp.zeros((pad_n,), jnp.int32).at[dest].set(tok)
    ws_slot = jnp.zeros((pad_n,), jnp.float32).at[dest].set(
        topk_weights.reshape(N)[order])
    invpos = jnp.zeros((N,), jnp.int32).at[order].set(dest)
    tile_starts = jnp.arange(tiles, dtype=jnp.int32) * M
    te = jnp.searchsorted(ends, tile_starts, side="right").astype(jnp.int32)
    last_e = jnp.searchsorted(ends, ends[-1] - 1, side="right").astype(jnp.int32)
    te = jnp.where(tile_starts < ends[-1], jnp.clip(te, 0, E - 1), last_e)
    nt = (ends[-1:] // M).astype(jnp.int32)
    return token_for_slot, ws_slot, invpos, te, nt


def _sc_gather(x, idx):
    pad_n = idx.shape[0]
    W = 32
    mesh = plsc.VectorSubcoreMesh(core_axis_name="core", subcore_axis_name="subcore")

    @functools.partial(
        pl.kernel,
        out_type=jax.ShapeDtypeStruct((pad_n, H), x.dtype),
        mesh=mesh,
    )
    def k(x_hbm, i_hbm, o_hbm):
        def body(i_vmem, o_vmem):
            pltpu.sync_copy(x_hbm.at[i_vmem.at[0]], o_vmem)

        pltpu.emit_pipeline(
            body,
            grid=(pad_n // W,),
            in_specs=[pl.BlockSpec((1, W), index_map=lambda i: (0, i))],
            out_specs=[pl.BlockSpec((W, H), index_map=lambda i: (i, 0))],
            core_axis_name=("core", "subcore"),
            dimension_semantics=(pltpu.PARALLEL,),
        )(i_hbm, o_hbm)

    return k(x, idx.reshape(1, pad_n))


def _sc_combine(ys, p0, p1):
    T_ = p0.shape[0]
    C = 16
    mesh = plsc.VectorSubcoreMesh(core_axis_name="core", subcore_axis_name="subcore")

    @functools.partial(
        pl.kernel,
        out_type=jax.ShapeDtypeStruct((T_, H), ys.dtype),
        mesh=mesh,
        scratch_types=[pltpu.VMEM((C, H), jnp.float32),
                       pltpu.VMEM((C, H), jnp.float32)],
    )
    def k(ys_hbm, p0_hbm, p1_hbm, o_hbm, a_v, b_v):
        def body(p0_v, p1_v, o_vmem):
            pltpu.sync_copy(ys_hbm.at[p0_v.at[0]], a_v)
            pltpu.sync_copy(ys_hbm.at[p1_v.at[0]], b_v)

            @pl.loop(0, C)
            def _(r):
                @pl.loop(0, H, step=16)
                def _(c):
                    o_vmem[r, pl.ds(c, 16)] = (a_v[r, pl.ds(c, 16)]
                                               + b_v[r, pl.ds(c, 16)])

        pltpu.emit_pipeline(
            body,
            grid=(T_ // C,),
            in_specs=[pl.BlockSpec((1, C), index_map=lambda i: (0, i)),
                      pl.BlockSpec((1, C), index_map=lambda i: (0, i))],
            out_specs=[pl.BlockSpec((C, H), index_map=lambda i: (i, 0))],
            core_axis_name=("core", "subcore"),
            dimension_semantics=(pltpu.PARALLEL,),
        )(p0_hbm, p1_hbm, o_hbm)

    return k(ys, p0.reshape(1, T_), p1.reshape(1, T_))


def _tc_gmm_body(te_ref, nt_ref, xs_ref, g_ref, u_ref, d_ref, ws_ref, ys_ref):
    i = pl.program_id(0)

    @pl.when(i < nt_ref[0])
    def _():
        x = xs_ref[...]
        gate = jnp.dot(x, g_ref[...], preferred_element_type=jnp.float32)
        up = jnp.dot(x, u_ref[...], preferred_element_type=jnp.float32)
        h = (gate * jax.nn.sigmoid(gate)) * up
        y = jnp.dot(h, d_ref[...], preferred_element_type=jnp.float32)
        w = ws_ref[0, 0, :]
        ys_ref[...] = y * w[:, None]


def _tc_gmm(xs, gate_up_proj, down_proj, ws3, te, nt, tiles, interpret=False):
    grid_spec = pltpu.PrefetchScalarGridSpec(
        num_scalar_prefetch=2,
        grid=(tiles,),
        in_specs=[
            pl.BlockSpec((M, H), lambda i, te, nt: (i, 0)),
            pl.BlockSpec((H, I), lambda i, te, nt: (te[i], 0)),
            pl.BlockSpec((H, I), lambda i, te, nt: (te[i], 1)),
            pl.BlockSpec((I, H), lambda i, te, nt: (te[i], 0)),
            pl.BlockSpec((1, 1, M), lambda i, te, nt: (i, 0, 0)),
        ],
        out_specs=pl.BlockSpec((M, H), lambda i, te, nt: (i, 0)),
    )
    return pl.pallas_call(
        _tc_gmm_body,
        grid_spec=grid_spec,
        out_shape=jax.ShapeDtypeStruct((tiles * M, H), jnp.float32),
        compiler_params=pltpu.CompilerParams(
            dimension_semantics=("arbitrary",)),
        interpret=interpret,
    )(te, nt, xs, gate_up_proj, gate_up_proj, down_proj, ws3)


def kernel(hidden_states, topk_weights, topk_indices, gate_up_proj, down_proj):
    T_, K_ = topk_indices.shape
    N = T_ * K_
    tiles = N // M + E
    token_for_slot, ws_slot, invpos, te, nt = _route(
        topk_indices, topk_weights, tiles)
    xs = _sc_gather(hidden_states, token_for_slot)
    ws3 = ws_slot.reshape(tiles, 1, M)
    ys = _tc_gmm(xs, gate_up_proj, down_proj, ws3, te, nt, tiles)
    pos = invpos.reshape(T_, K_)
    out = _sc_combine(ys, pos[:, 0], pos[:, 1])
    return out.astype(hidden_states.dtype)


# speedup vs baseline: 1.2789x; 1.2789x over previous
import functools

import jax
import jax.numpy as jnp
from jax import lax
from jax.experimental import pallas as pl
from jax.experimental.pallas import tpu as pltpu
from jax.experimental.pallas import tpu_sc as plsc

NC = 2
NS = 16
NW = NC * NS

E = 16
H = 1024
I = 1024
M = 256


def _route(topk_indices, topk_weights, tiles):
    T_, K_ = topk_indices.shape
    N = T_ * K_
    pad_n = tiles * M
    e_flat = topk_indices.reshape(N).astype(jnp.int32)
    order = jnp.argsort(e_flat, stable=True).astype(jnp.int32)
    sorted_e = e_flat[order]
    g = jnp.zeros((E,), jnp.int32).at[e_flat].add(1)
    seg_start = (jnp.cumsum(g) - g).astype(jnp.int32)
    padded = ((g + (M - 1)) // M) * M
    ends = jnp.cumsum(padded).astype(jnp.int32)
    base = ends - padded
    s = jnp.arange(N, dtype=jnp.int32)
    rank = s - seg_start[sorted_e]
    dest = base[sorted_e] + rank
    tok = order // K_
    token_for_slot = jnp.zeros((pad_n,), jnp.int32).at[dest].set(tok)
    ws_slot = jnp.zeros((pad_n,), jnp.float32).at[dest].set(
        topk_weights.reshape(N)[order])
    invpos = jnp.zeros((N,), jnp.int32).at[order].set(dest)
    tile_starts = jnp.arange(tiles, dtype=jnp.int32) * M
    te = jnp.searchsorted(ends, tile_starts, side="right").astype(jnp.int32)
    last_e = jnp.searchsorted(ends, ends[-1] - 1, side="right").astype(jnp.int32)
    te = jnp.where(tile_starts < ends[-1], jnp.clip(te, 0, E - 1), last_e)
    nt = (ends[-1:] // M).astype(jnp.int32)
    return token_for_slot, ws_slot, invpos, te, nt


def _sc_gather(x, idx):
    pad_n = idx.shape[0]
    per_w = pad_n // NW
    W = 32
    nch = per_w // W
    mesh = plsc.VectorSubcoreMesh(core_axis_name="core", subcore_axis_name="subcore")

    @functools.partial(
        pl.kernel,
        out_type=jax.ShapeDtypeStruct((pad_n, H), x.dtype),
        mesh=mesh,
        scratch_types=[pltpu.VMEM((per_w,), jnp.int32),
                       pltpu.VMEM((W, H), jnp.float32),
                       pltpu.VMEM((W, H), jnp.float32),
                       pltpu.SemaphoreType.DMA,
                       pltpu.SemaphoreType.DMA],
    )
    def k(x_hbm, i_hbm, o_hbm, idx_v, r0, r1, s0, s1):
        wid = lax.axis_index("subcore") * NC + lax.axis_index("core")
        base = wid * per_w
        pltpu.sync_copy(i_hbm.at[pl.ds(base, per_w)], idx_v)
        bufs = [(r0, s0), (r1, s1)]
        pltpu.async_copy(x_hbm.at[idx_v.at[pl.ds(0, W)]], r0, s0)
        for c in range(nch):
            r, s = bufs[c % 2]
            pltpu.make_async_copy(x_hbm.at[idx_v.at[pl.ds(c * W, W)]], r, s).wait()
            if c + 1 < nch:
                rn, sn = bufs[(c + 1) % 2]
                pltpu.async_copy(
                    x_hbm.at[idx_v.at[pl.ds((c + 1) * W, W)]], rn, sn)
            pltpu.sync_copy(r, o_hbm.at[pl.ds(base + c * W, W)])

    return k(x, idx)


def _sc_combine(ys, p0, p1):
    T_ = p0.shape[0]
    per_w = T_ // NW
    C = 16
    nch = per_w // C
    mesh = plsc.VectorSubcoreMesh(core_axis_name="core", subcore_axis_name="subcore")

    @functools.partial(
        pl.kernel,
        out_type=jax.ShapeDtypeStruct((T_, H), ys.dtype),
        mesh=mesh,
        scratch_types=[pltpu.VMEM((per_w,), jnp.int32),
                       pltpu.VMEM((per_w,), jnp.int32),
                       pltpu.VMEM((C, H), jnp.float32),
                       pltpu.VMEM((C, H), jnp.float32),
                       pltpu.VMEM((C, H), jnp.float32),
                       pltpu.SemaphoreType.DMA,
                       pltpu.SemaphoreType.DMA],
    )
    def k(ys_hbm, p0_hbm, p1_hbm, o_hbm, p0_v, p1_v, a_v, b_v, o_v, s0, s1):
        wid = lax.axis_index("subcore") * NC + lax.axis_index("core")
        base = wid * per_w
        pltpu.sync_copy(p0_hbm.at[pl.ds(base, per_w)], p0_v)
        pltpu.sync_copy(p1_hbm.at[pl.ds(base, per_w)], p1_v)
        for c in range(nch):
            pltpu.async_copy(ys_hbm.at[p0_v.at[pl.ds(c * C, C)]], a_v, s0)
            pltpu.async_copy(ys_hbm.at[p1_v.at[pl.ds(c * C, C)]], b_v, s1)
            pltpu.make_async_copy(ys_hbm.at[p0_v.at[pl.ds(c * C, C)]], a_v, s0).wait()
            pltpu.make_async_copy(ys_hbm.at[p1_v.at[pl.ds(c * C, C)]], b_v, s1).wait()

            @pl.loop(0, C)
            def _(r):
                @pl.loop(0, H, step=16)
                def _(col):
                    o_v[r, pl.ds(col, 16)] = (a_v[r, pl.ds(col, 16)]
                                              + b_v[r, pl.ds(col, 16)])

            pltpu.sync_copy(o_v, o_hbm.at[pl.ds(base + c * C, C)])

    return k(ys, p0, p1)


def _tc_gmm_body(te_ref, nt_ref, xs_ref, g_ref, u_ref, d_ref, ws_ref, ys_ref):
    i = pl.program_id(0)

    @pl.when(i < nt_ref[0])
    def _():
        x = xs_ref[...]
        gate = jnp.dot(x, g_ref[...], preferred_element_type=jnp.float32)
        up = jnp.dot(x, u_ref[...], preferred_element_type=jnp.float32)
        h = (gate * jax.nn.sigmoid(gate)) * up
        y = jnp.dot(h, d_ref[...], preferred_element_type=jnp.float32)
        w = ws_ref[0, 0, :]
        ys_ref[...] = y * w[:, None]


def _tc_gmm(xs, gate_up_proj, down_proj, ws3, te, nt, tiles, interpret=False):
    grid_spec = pltpu.PrefetchScalarGridSpec(
        num_scalar_prefetch=2,
        grid=(tiles,),
        in_specs=[
            pl.BlockSpec((M, H), lambda i, te, nt: (i, 0)),
            pl.BlockSpec((H, I), lambda i, te, nt: (te[i], 0)),
            pl.BlockSpec((H, I), lambda i, te, nt: (te[i], 1)),
            pl.BlockSpec((I, H), lambda i, te, nt: (te[i], 0)),
            pl.BlockSpec((1, 1, M), lambda i, te, nt: (i, 0, 0)),
        ],
        out_specs=pl.BlockSpec((M, H), lambda i, te, nt: (i, 0)),
    )
    return pl.pallas_call(
        _tc_gmm_body,
        grid_spec=grid_spec,
        out_shape=jax.ShapeDtypeStruct((tiles * M, H), jnp.float32),
        compiler_params=pltpu.CompilerParams(
            dimension_semantics=("arbitrary",)),
        interpret=interpret,
    )(te, nt, xs, gate_up_proj, gate_up_proj, down_proj, ws3)


def kernel(hidden_states, topk_weights, topk_indices, gate_up_proj, down_proj):
    T_, K_ = topk_indices.shape
    N = T_ * K_
    tiles = N // M + E
    token_for_slot, ws_slot, invpos, te, nt = _route(
        topk_indices, topk_weights, tiles)
    xs = _sc_gather(hidden_states, token_for_slot)
    ws3 = ws_slot.reshape(tiles, 1, M)
    ys = _tc_gmm(xs, gate_up_proj, down_proj, ws3, te, nt, tiles)
    pos = invpos.reshape(T_, K_)
    out = _sc_combine(ys, pos[:, 0] + 0, pos[:, 1] + 0)
    return out.astype(hidden_states.dtype)
